# Initial kernel scaffold; baseline (speedup 1.0000x reference)
import jax
import jax.numpy as jnp
from jax.experimental import pallas as pl


def kernel(voc_idxs, pai_concept, pai_character):
    raise NotImplementedError("write your pallas kernel here")


# speedup vs baseline: 2.6457x; 2.6457x over previous
import functools

import jax
import jax.numpy as jnp
from jax import lax
from jax.experimental import pallas as pl
from jax.experimental.pallas import tpu as pltpu
from jax.experimental.pallas import tpu_sc as plsc

D = 16
K = 32
NC, NS, L = 2, 16, 16
NW = NC * NS


def kernel(voc_idxs, pai_concept, pai_character):
    B = voc_idxs.shape[0]
    BPW = B // NW
    CH = 64
    NCHUNK = BPW // CH

    idx3 = voc_idxs.reshape(NW, NCHUNK, CH)
    pc2 = pai_concept.reshape(-1, D * K)
    ph2 = pai_character.reshape(-1, D * K)
    mesh = plsc.VectorSubcoreMesh(core_axis_name="c", subcore_axis_name="s")

    @functools.partial(
        pl.kernel,
        out_type=jax.ShapeDtypeStruct((B, D), jnp.int32),
        mesh=mesh,
        scratch_types=[
            pltpu.VMEM((NCHUNK, CH), jnp.int32),
            pltpu.VMEM((CH, D * K), jnp.float32),
            pltpu.VMEM((CH, D * K), jnp.float32),
            pltpu.VMEM((BPW, D), jnp.int32),
            pltpu.SemaphoreType.DMA,
            pltpu.SemaphoreType.DMA,
        ],
        compiler_params=pltpu.CompilerParams(needs_layout_passes=False),
    )
    def sc_kernel(idx_hbm, pc_hbm, ph_hbm, out_hbm,
                  idx_v, rows_c, rows_h, out_v, sem_c, sem_h):
        cid = lax.axis_index("c")
        sid = lax.axis_index("s")
        wid = sid * NC + cid
        pltpu.sync_copy(idx_hbm.at[wid], idx_v)

        lanes = lax.iota(jnp.int32, L)
        neg_inf = jnp.full((L,), -jnp.inf, jnp.float32)
        zero = jnp.zeros((L,), jnp.int32)

        for ci in range(NCHUNK):
            cpy_c = pltpu.async_copy(pc_hbm.at[idx_v.at[ci]], rows_c, sem_c)
            cpy_h = pltpu.async_copy(ph_hbm.at[idx_v.at[ci]], rows_h, sem_h)
            cpy_c.wait()
            cpy_h.wait()
            for g in range(CH // L):
                ib = jnp.full((L,), g * L, jnp.int32) + lanes
                ob = jnp.full((L,), ci * CH + g * L, jnp.int32) + lanes

                def d_body(d, _, ib=ib, ob=ob):
                    id_ = jnp.full((L,), d, jnp.int32)
                    col0 = jnp.full((L,), d * K, jnp.int32)

                    def k_body(k, carry):
                        m_c, a_c, m_h, a_h = carry
                        ik = jnp.full((L,), k, jnp.int32)
                        ic = col0 + ik
                        xc = plsc.load_gather(rows_c, [ib, ic])
                        xh = plsc.load_gather(rows_h, [ib, ic])
                        gt_c = xc > m_c
                        gt_h = xh > m_h
                        return (jnp.where(gt_c, xc, m_c),
                                jnp.where(gt_c, ik, a_c),
                                jnp.where(gt_h, xh, m_h),
                                jnp.where(gt_h, ik, a_h))

                    m_c, a_c, m_h, a_h = lax.fori_loop(
                        0, K, k_body, (neg_inf, zero, neg_inf, zero))
                    plsc.store_scatter(out_v, [ob, id_], a_c + a_h)
                    return _

                lax.fori_loop(0, D, d_body, 0)

        pltpu.sync_copy(out_v, out_hbm.at[pl.ds(wid * BPW, BPW)])

    return sc_kernel(idx3, pc2, ph2)


# speedup vs baseline: 2.6646x; 1.0071x over previous
import functools

import jax
import jax.numpy as jnp
from jax import lax
from jax.experimental import pallas as pl
from jax.experimental.pallas import tpu as pltpu
from jax.experimental.pallas import tpu_sc as plsc

D = 16
K = 32
NC, NS, L = 2, 16, 16
NW = NC * NS
NBUF = 2


def _argmax_tree(pairs):
    if len(pairs) == 1:
        return pairs[0]
    mid = len(pairs) // 2
    ml, al = _argmax_tree(pairs[:mid])
    mr, ar = _argmax_tree(pairs[mid:])
    keep = ml >= mr
    return jnp.where(keep, ml, mr), jnp.where(keep, al, ar)


def kernel(voc_idxs, pai_concept, pai_character):
    B = voc_idxs.shape[0]
    BPW = B // NW
    CH = 32
    NCHUNK = BPW // CH

    idx3 = voc_idxs.reshape(NW, NCHUNK, CH)
    pc2 = pai_concept.reshape(-1, D * K)
    ph2 = pai_character.reshape(-1, D * K)
    mesh = plsc.VectorSubcoreMesh(core_axis_name="c", subcore_axis_name="s")

    @functools.partial(
        pl.kernel,
        out_type=jax.ShapeDtypeStruct((B, D), jnp.int32),
        mesh=mesh,
        scratch_types=[
            pltpu.VMEM((NCHUNK, CH), jnp.int32),
            pltpu.VMEM((NBUF, CH, D * K), jnp.float32),
            pltpu.VMEM((NBUF, CH, D * K), jnp.float32),
            pltpu.VMEM((BPW, D), jnp.int32),
            pltpu.SemaphoreType.DMA((NBUF,)),
            pltpu.SemaphoreType.DMA((NBUF,)),
        ],
        compiler_params=pltpu.CompilerParams(needs_layout_passes=False),
    )
    def sc_kernel(idx_hbm, pc_hbm, ph_hbm, out_hbm,
                  idx_v, rows_c, rows_h, out_v, sem_c, sem_h):
        cid = lax.axis_index("c")
        sid = lax.axis_index("s")
        wid = sid * NC + cid
        pltpu.sync_copy(idx_hbm.at[wid], idx_v)

        lanes = lax.iota(jnp.int32, L)
        cols = [lanes * K + k for k in range(K)]

        def start(ci):
            bi = ci % NBUF
            c = pltpu.async_copy(pc_hbm.at[idx_v.at[ci]], rows_c.at[bi],
                                 sem_c.at[bi])
            h = pltpu.async_copy(ph_hbm.at[idx_v.at[ci]], rows_h.at[bi],
                                 sem_h.at[bi])
            return c, h

        pending = {0: start(0)}
        for ci in range(NCHUNK):
            if ci + 1 < NCHUNK:
                pending[ci + 1] = start(ci + 1)
            c, h = pending.pop(ci)
            c.wait()
            h.wait()
            bi = ci % NBUF
            buf_c = rows_c.at[bi]
            buf_h = rows_h.at[bi]

            def b_body(b, _, ci=ci, buf_c=buf_c, buf_h=buf_h):
                ibs = jnp.full((L,), b, jnp.int32)

                def table_argmax(buf):
                    pairs = [
                        (plsc.load_gather(buf, [ibs, cols[k]]),
                         jnp.full((L,), k, jnp.int32))
                        for k in range(K)
                    ]
                    return _argmax_tree(pairs)[1]

                a = table_argmax(buf_c) + table_argmax(buf_h)
                out_v[ci * CH + b, :] = a
                return _

            lax.fori_loop(0, CH, b_body, 0)

        pltpu.sync_copy(out_v, out_hbm.at[pl.ds(wid * BPW, BPW)])

    return sc_kernel(idx3, pc2, ph2)


# speedup vs baseline: 4.2000x; 1.5762x over previous
import functools

import jax
import jax.numpy as jnp
from jax import lax
from jax.experimental import pallas as pl
from jax.experimental.pallas import tpu as pltpu
from jax.experimental.pallas import tpu_sc as plsc

D = 16
K = 32
NC, NS, L = 2, 16, 16
NBUF = 2
KG = 8
KH = K // 2
NKG = KH // KG
VC = 4096
VMAIN = 99968
NCH = 25
VLAST = VMAIN - VC


def kernel(voc_idxs, pai_concept, pai_character):
    B = voc_idxs.shape[0]
    V = pai_concept.shape[0]
    TAIL = V - VMAIN

    vt_c = pai_concept.transpose(1, 2, 0)
    vt_h = pai_character.transpose(1, 2, 0)
    tails = jnp.stack([vt_c[:, :, VMAIN:], vt_h[:, :, VMAIN:]])

    order = jnp.argsort(voc_idxs).astype(jnp.int32)
    sv = voc_idxs[order]
    grid = jnp.minimum(jnp.arange(NCH, dtype=jnp.int32) * VC, VLAST)
    los = jnp.searchsorted(sv, grid).astype(jnp.int32)
    his = jnp.searchsorted(sv, grid + VC).astype(jnp.int32)
    lo_tail = jnp.searchsorted(sv, jnp.int32(VMAIN)).astype(jnp.int32)
    bounds = jnp.concatenate(
        [los, his, lo_tail[None], jnp.full((64 - 2 * NCH - 1,), B,
                                           jnp.int32)])

    mesh = plsc.VectorSubcoreMesh(core_axis_name="c", subcore_axis_name="s")

    @functools.partial(
        pl.kernel,
        out_type=jax.ShapeDtypeStruct((D, B), jnp.int32),
        mesh=mesh,
        scratch_types=[
            pltpu.VMEM((64,), jnp.int32),
            pltpu.VMEM((B,), jnp.int32),
            pltpu.VMEM((B,), jnp.int32),
            pltpu.VMEM((NBUF * KG, VC), jnp.float32),
            pltpu.VMEM((K, TAIL), jnp.float32),
            pltpu.VMEM((2, B), jnp.float32),
            pltpu.VMEM((2, B), jnp.int32),
            pltpu.VMEM((B,), jnp.float32),
            pltpu.VMEM((B,), jnp.int32),
            pltpu.VMEM((B,), jnp.int32),
            pltpu.VMEM_SHARED((NS, 2, B), jnp.float32),
            pltpu.VMEM_SHARED((NS, 2, B), jnp.int32),
            pltpu.SemaphoreType.DMA((NBUF,)),
        ],
        compiler_params=pltpu.CompilerParams(needs_layout_passes=False),
    )
    def sc_kernel(sv_hbm, order_hbm, bounds_hbm, c_hbm, h_hbm, tails_hbm,
                  out_hbm, bnd_v, sv_v, ord_v, blk_v, tail_v,
                  m_v, a_v, pm_v, pa_v, row_v, shf_v, shi_v, sem):
        cid = lax.axis_index("c")
        sid = lax.axis_index("s")
        d = (sid // 2) * NC + cid
        kh = sid % 2
        k0 = kh * KH

        pltpu.sync_copy(bounds_hbm, bnd_v)
        pltpu.sync_copy(sv_hbm, sv_v)
        pltpu.sync_copy(order_hbm, ord_v)

        neg_inf = jnp.full((L,), -jnp.inf, jnp.float32)
        zeros = jnp.zeros((L,), jnp.int32)

        def init_body(i, _):
            s = pl.ds(i * L, L)
            for tb in range(2):
                m_v[tb, s] = neg_inf
                a_v[tb, s] = zeros
            return 0

        lax.fori_loop(0, B // L, init_body, 0)

        def bscal(i):
            return plsc.load_gather(bnd_v, [jnp.full((L,), i, jnp.int32)])[0]

        def issue(tab_hbm, o, c, bi):
            v0 = pl.multiple_of(jnp.minimum(c * VC, VLAST), 128)
            row0 = pl.multiple_of(bi * KG, KG)
            krow = pl.multiple_of(k0 + o * KG, KG)
            pltpu.async_copy(
                tab_hbm.at[d, pl.ds(krow, KG), pl.ds(v0, VC)],
                blk_v.at[pl.ds(row0, KG)], sem.at[bi])

        def drain(bi):
            dummy = c_hbm.at[0, pl.ds(0, KG), pl.ds(0, VC)]
            pltpu.make_async_copy(dummy, blk_v.at[pl.ds(0, KG)],
                                  sem.at[bi]).wait()

        def scan_chunk(tb, o, c, bi):
            v0s = jnp.broadcast_to(
                jnp.minimum(c * VC, VLAST), (L,)).astype(jnp.int32)
            lo16 = bscal(c) // L
            hi16 = (bscal(NCH + c) + L - 1) // L

            def vec_body(vec, _):
                s = pl.ds(vec * L, L)
                vb = sv_v[s]
                off = vb - v0s
                inb = (off >= 0) & (off < VC)
                offc = jnp.clip(off, 0, VC - 1)
                m16 = m_v[tb, s]
                a16 = a_v[tb, s]
                rowb = jnp.broadcast_to(bi * KG, (L,)).astype(jnp.int32)
                kkb = jnp.broadcast_to(k0 + o * KG, (L,)).astype(jnp.int32)
                for i in range(KG):
                    row = rowb + jnp.full((L,), i, jnp.int32)
                    x = plsc.load_gather(blk_v, [row, offc])
                    upd = (x > m16) & inb
                    m16 = jnp.where(upd, x, m16)
                    a16 = jnp.where(upd, kkb + jnp.full((L,), i, jnp.int32),
                                    a16)
                m_v[tb, s] = m16
                a_v[tb, s] = a16
                return 0

            lax.fori_loop(lo16, hi16, vec_body, 0)

        def scan_tail(tb, o):
            v0s = jnp.full((L,), VMAIN, jnp.int32)
            lo16 = bscal(2 * NCH) // L
            hi16 = (bscal(2 * NCH + 1) + L - 1) // L

            def vec_body(vec, _):
                s = pl.ds(vec * L, L)
                vb = sv_v[s]
                off = vb - v0s
                inb = off >= 0
                offc = jnp.clip(off, 0, TAIL - 1)
                m16 = m_v[tb, s]
                a16 = a_v[tb, s]
                kkb = jnp.broadcast_to(k0 + o * KG, (L,)).astype(jnp.int32)
                for i in range(KG):
                    krow = kkb + jnp.full((L,), i, jnp.int32)
                    x = plsc.load_gather(tail_v, [krow, offc])
                    upd = (x > m16) & inb
                    m16 = jnp.where(upd, x, m16)
                    a16 = jnp.where(upd, krow, a16)
                m_v[tb, s] = m16
                a_v[tb, s] = a16
                return 0

            lax.fori_loop(lo16, hi16, vec_body, 0)

        for tb, tab_hbm in ((0, c_hbm), (1, h_hbm)):
            pltpu.sync_copy(tails_hbm.at[tb, d], tail_v)
            for o in range(NKG):
                issue(tab_hbm, o, jnp.int32(0), jnp.int32(0))

                def chunk_body(c, _, tb=tb, o=o, tab_hbm=tab_hbm):
                    bi = c % 2

                    @pl.when(c + 1 < NCH)
                    def _():
                        issue(tab_hbm, o, c + 1, (c + 1) % 2)

                    drain(bi)
                    scan_chunk(tb, o, c, bi)
                    return 0

                lax.fori_loop(0, NCH, chunk_body, 0)
                scan_tail(tb, o)

        pltpu.sync_copy(m_v, shf_v.at[sid])
        pltpu.sync_copy(a_v, shi_v.at[sid])
        plsc.subcore_barrier()

        @pl.when(kh == 0)
        def _():
            for tb in range(2):
                pltpu.sync_copy(shf_v.at[sid + 1, tb], pm_v)
                pltpu.sync_copy(shi_v.at[sid + 1, tb], pa_v)

                def merge_body(i, _, tb=tb):
                    s = pl.ds(i * L, L)
                    upd = pm_v[s] > m_v[tb, s]
                    a_v[tb, s] = jnp.where(upd, pa_v[s], a_v[tb, s])
                    return 0

                lax.fori_loop(0, B // L, merge_body, 0)

            def out_body(i, _):
                s = pl.ds(i * L, L)
                plsc.store_scatter(row_v, [ord_v[s]], a_v[0, s] + a_v[1, s])
                return 0

            lax.fori_loop(0, B // L, out_body, 0)
            pltpu.sync_copy(row_v, out_hbm.at[d])

    out_t = sc_kernel(sv, order, bounds, vt_c, vt_h, tails)
    return out_t.T


# speedup vs baseline: 5.2004x; 1.2382x over previous
import functools

import jax
import jax.numpy as jnp
from jax import lax
from jax.experimental import pallas as pl
from jax.experimental.pallas import tpu as pltpu
from jax.experimental.pallas import tpu_sc as plsc

D = 16
K = 32
NC, NS, L = 2, 16, 16
NBUF = 2
KG = 8
KH = K // 2
NKG = KH // KG
VC = 4096
VMAIN = 99968
NCH = 25
VLAST = VMAIN - VC


def kernel(voc_idxs, pai_concept, pai_character):
    B = voc_idxs.shape[0]
    V = pai_concept.shape[0]
    TAIL = V - VMAIN

    vt_c = pai_concept.transpose(1, 2, 0)
    vt_h = pai_character.transpose(1, 2, 0)
    tails = jnp.stack([vt_c[:, :, VMAIN:], vt_h[:, :, VMAIN:]])

    order = jnp.argsort(voc_idxs).astype(jnp.int32)
    sv = voc_idxs[order]
    grid = jnp.minimum(jnp.arange(NCH, dtype=jnp.int32) * VC, VLAST)
    cuts = jnp.concatenate([grid, grid + VC, jnp.array([VMAIN], jnp.int32),
                            jnp.full((64 - 2 * NCH - 1,), V, jnp.int32)])
    bounds = jnp.sum(sv[None, :] < cuts[:, None], axis=1,
                     dtype=jnp.int32)

    mesh = plsc.VectorSubcoreMesh(core_axis_name="c", subcore_axis_name="s")

    @functools.partial(
        pl.kernel,
        out_type=jax.ShapeDtypeStruct((D, B), jnp.int32),
        mesh=mesh,
        scratch_types=[
            pltpu.VMEM((64,), jnp.int32),
            pltpu.VMEM((B,), jnp.int32),
            pltpu.VMEM((B,), jnp.int32),
            pltpu.VMEM((NBUF * KG, VC), jnp.float32),
            pltpu.VMEM((K, TAIL), jnp.float32),
            pltpu.VMEM((2, B), jnp.float32),
            pltpu.VMEM((2, B), jnp.int32),
            pltpu.VMEM((B,), jnp.float32),
            pltpu.VMEM((B,), jnp.int32),
            pltpu.VMEM((B,), jnp.int32),
            pltpu.VMEM_SHARED((NS, 2, B), jnp.float32),
            pltpu.VMEM_SHARED((NS, 2, B), jnp.int32),
            pltpu.SemaphoreType.DMA((NBUF,)),
        ],
        compiler_params=pltpu.CompilerParams(needs_layout_passes=False),
    )
    def sc_kernel(sv_hbm, order_hbm, bounds_hbm, c_hbm, h_hbm, tails_hbm,
                  out_hbm, bnd_v, sv_v, ord_v, blk_v, tail_v,
                  m_v, a_v, pm_v, pa_v, row_v, shf_v, shi_v, sem):
        cid = lax.axis_index("c")
        sid = lax.axis_index("s")
        d = (sid // 2) * NC + cid
        kh = sid % 2
        k0 = kh * KH

        pltpu.sync_copy(bounds_hbm, bnd_v)
        pltpu.sync_copy(sv_hbm, sv_v)
        pltpu.sync_copy(order_hbm, ord_v)

        neg_inf = jnp.full((L,), -jnp.inf, jnp.float32)
        zeros = jnp.zeros((L,), jnp.int32)

        def init_body(i, _):
            s = pl.ds(i * L, L)
            for tb in range(2):
                m_v[tb, s] = neg_inf
                a_v[tb, s] = zeros
            return 0

        lax.fori_loop(0, B // L, init_body, 0)

        def bscal(i):
            return plsc.load_gather(bnd_v, [jnp.full((L,), i, jnp.int32)])[0]

        def issue(tab_hbm, o, c, bi):
            v0 = pl.multiple_of(jnp.minimum(c * VC, VLAST), 128)
            row0 = pl.multiple_of(bi * KG, KG)
            krow = pl.multiple_of(k0 + o * KG, KG)
            pltpu.async_copy(
                tab_hbm.at[d, pl.ds(krow, KG), pl.ds(v0, VC)],
                blk_v.at[pl.ds(row0, KG)], sem.at[bi])

        def drain(bi):
            dummy = c_hbm.at[0, pl.ds(0, KG), pl.ds(0, VC)]
            pltpu.make_async_copy(dummy, blk_v.at[pl.ds(0, KG)],
                                  sem.at[bi]).wait()

        def scan_chunk(tb, o, c, bi):
            v0s = jnp.broadcast_to(
                jnp.minimum(c * VC, VLAST), (L,)).astype(jnp.int32)
            lo16 = bscal(c) // L
            hi16 = (bscal(NCH + c) + L - 1) // L

            def vec_body(vec, _):
                s = pl.ds(vec * L, L)
                vb = sv_v[s]
                off = vb - v0s
                inb = (off >= 0) & (off < VC)
                offc = jnp.clip(off, 0, VC - 1)
                m16 = m_v[tb, s]
                a16 = a_v[tb, s]
                rowb = jnp.broadcast_to(bi * KG, (L,)).astype(jnp.int32)
                kkb = jnp.broadcast_to(k0 + o * KG, (L,)).astype(jnp.int32)
                for i in range(KG):
                    row = rowb + jnp.full((L,), i, jnp.int32)
                    x = plsc.load_gather(blk_v, [row, offc])
                    upd = (x > m16) & inb
                    m16 = jnp.where(upd, x, m16)
                    a16 = jnp.where(upd, kkb + jnp.full((L,), i, jnp.int32),
                                    a16)
                m_v[tb, s] = m16
                a_v[tb, s] = a16
                return 0

            lax.fori_loop(lo16, hi16, vec_body, 0)

        def scan_tail(tb, o):
            v0s = jnp.full((L,), VMAIN, jnp.int32)
            lo16 = bscal(2 * NCH) // L
            hi16 = (bscal(2 * NCH + 1) + L - 1) // L

            def vec_body(vec, _):
                s = pl.ds(vec * L, L)
                vb = sv_v[s]
                off = vb - v0s
                inb = off >= 0
                offc = jnp.clip(off, 0, TAIL - 1)
                m16 = m_v[tb, s]
                a16 = a_v[tb, s]
                kkb = jnp.broadcast_to(k0 + o * KG, (L,)).astype(jnp.int32)
                for i in range(KG):
                    krow = kkb + jnp.full((L,), i, jnp.int32)
                    x = plsc.load_gather(tail_v, [krow, offc])
                    upd = (x > m16) & inb
                    m16 = jnp.where(upd, x, m16)
                    a16 = jnp.where(upd, krow, a16)
                m_v[tb, s] = m16
                a_v[tb, s] = a16
                return 0

            lax.fori_loop(lo16, hi16, vec_body, 0)

        for tb, tab_hbm in ((0, c_hbm), (1, h_hbm)):
            pltpu.sync_copy(tails_hbm.at[tb, d], tail_v)
            for o in range(NKG):
                issue(tab_hbm, o, jnp.int32(0), jnp.int32(0))

                def chunk_body(c, _, tb=tb, o=o, tab_hbm=tab_hbm):
                    bi = c % 2

                    @pl.when(c + 1 < NCH)
                    def _():
                        issue(tab_hbm, o, c + 1, (c + 1) % 2)

                    drain(bi)
                    scan_chunk(tb, o, c, bi)
                    return 0

                lax.fori_loop(0, NCH, chunk_body, 0)
                scan_tail(tb, o)

        pltpu.sync_copy(m_v, shf_v.at[sid])
        pltpu.sync_copy(a_v, shi_v.at[sid])
        plsc.subcore_barrier()

        @pl.when(kh == 0)
        def _():
            for tb in range(2):
                pltpu.sync_copy(shf_v.at[sid + 1, tb], pm_v)
                pltpu.sync_copy(shi_v.at[sid + 1, tb], pa_v)

                def merge_body(i, _, tb=tb):
                    s = pl.ds(i * L, L)
                    upd = pm_v[s] > m_v[tb, s]
                    a_v[tb, s] = jnp.where(upd, pa_v[s], a_v[tb, s])
                    return 0

                lax.fori_loop(0, B // L, merge_body, 0)

            def out_body(i, _):
                s = pl.ds(i * L, L)
                plsc.store_scatter(row_v, [ord_v[s]], a_v[0, s] + a_v[1, s])
                return 0

            lax.fori_loop(0, B // L, out_body, 0)
            pltpu.sync_copy(row_v, out_hbm.at[d])

    out_t = sc_kernel(sv, order, bounds, vt_c, vt_h, tails)
    return out_t.T


# speedup vs baseline: 5.4486x; 1.0477x over previous
import functools

import jax
import jax.numpy as jnp
from jax import lax
from jax.experimental import pallas as pl
from jax.experimental.pallas import tpu as pltpu
from jax.experimental.pallas import tpu_sc as plsc

D = 16
K = 32
NC, NS, L = 2, 16, 16
NBUF = 2
KG = 8
KH = K // 2
NKG = KH // KG
VC = 4096
VMAIN = 99968
NCH = 25
VLAST = VMAIN - VC


def kernel(voc_idxs, pai_concept, pai_character):
    B = voc_idxs.shape[0]
    V = pai_concept.shape[0]
    TAIL = V - VMAIN

    vt_c = pai_concept.transpose(1, 2, 0)
    vt_h = pai_character.transpose(1, 2, 0)
    tails = jnp.stack([vt_c[:, :, VMAIN:], vt_h[:, :, VMAIN:]])

    order = jnp.argsort(voc_idxs).astype(jnp.int32)
    sv = voc_idxs[order]
    grid = jnp.minimum(jnp.arange(NCH, dtype=jnp.int32) * VC, VLAST)
    cuts = jnp.concatenate([grid, grid + VC, jnp.array([VMAIN], jnp.int32),
                            jnp.full((64 - 2 * NCH - 1,), V, jnp.int32)])
    bounds = jnp.sum(sv[None, :] < cuts[:, None], axis=1,
                     dtype=jnp.int32)

    mesh = plsc.VectorSubcoreMesh(core_axis_name="c", subcore_axis_name="s")

    @functools.partial(
        pl.kernel,
        out_type=jax.ShapeDtypeStruct((D, B), jnp.int32),
        mesh=mesh,
        scratch_types=[
            pltpu.VMEM((64,), jnp.int32),
            pltpu.VMEM((B,), jnp.int32),
            pltpu.VMEM((B,), jnp.int32),
            pltpu.VMEM((NBUF * KG, VC), jnp.float32),
            pltpu.VMEM((K, TAIL), jnp.float32),
            pltpu.VMEM((2, B), jnp.float32),
            pltpu.VMEM((2, B), jnp.int32),
            pltpu.VMEM((B,), jnp.float32),
            pltpu.VMEM((B,), jnp.int32),
            pltpu.VMEM((B,), jnp.int32),
            pltpu.VMEM_SHARED((NS, 2, B), jnp.float32),
            pltpu.VMEM_SHARED((NS, 2, B), jnp.int32),
            pltpu.SemaphoreType.DMA((NBUF,)),
            pltpu.SemaphoreType.DMA((NBUF,)),
        ],
        compiler_params=pltpu.CompilerParams(needs_layout_passes=False),
    )
    def sc_kernel(sv_hbm, order_hbm, bounds_hbm, c_hbm, h_hbm, tails_hbm,
                  out_hbm, bnd_v, sv_v, ord_v, blk_v, tail_v,
                  m_v, a_v, pm_v, pa_v, row_v, shf_v, shi_v, sem, sem2):
        cid = lax.axis_index("c")
        sid = lax.axis_index("s")
        d = (sid // 2) * NC + cid
        kh = sid % 2
        k0 = kh * KH

        pltpu.sync_copy(bounds_hbm, bnd_v)
        pltpu.sync_copy(sv_hbm, sv_v)
        pltpu.sync_copy(order_hbm, ord_v)

        neg_inf = jnp.full((L,), -jnp.inf, jnp.float32)
        zeros = jnp.zeros((L,), jnp.int32)

        def init_body(i, _):
            s = pl.ds(i * L, L)
            for tb in range(2):
                m_v[tb, s] = neg_inf
                a_v[tb, s] = zeros
            return 0

        lax.fori_loop(0, B // L, init_body, 0)

        def bscal(i):
            return plsc.load_gather(bnd_v, [jnp.full((L,), i, jnp.int32)])[0]

        HV = VC // 2

        def issue(tab_hbm, o, c, bi):
            v0 = pl.multiple_of(jnp.minimum(c * VC, VLAST), 128)
            v1 = pl.multiple_of(v0 + HV, 128)
            row0 = pl.multiple_of(bi * KG, KG)
            krow = pl.multiple_of(k0 + o * KG, KG)
            pltpu.async_copy(
                tab_hbm.at[d, pl.ds(krow, KG), pl.ds(v0, HV)],
                blk_v.at[pl.ds(row0, KG), pl.ds(0, HV)], sem.at[bi])
            pltpu.async_copy(
                tab_hbm.at[d, pl.ds(krow, KG), pl.ds(v1, HV)],
                blk_v.at[pl.ds(row0, KG), pl.ds(HV, HV)], sem2.at[bi])

        def drain(bi):
            dummy = c_hbm.at[0, pl.ds(0, KG), pl.ds(0, HV)]
            pltpu.make_async_copy(dummy, blk_v.at[pl.ds(0, KG), pl.ds(0, HV)],
                                  sem.at[bi]).wait()
            pltpu.make_async_copy(dummy, blk_v.at[pl.ds(0, KG), pl.ds(0, HV)],
                                  sem2.at[bi]).wait()

        def scan_chunk(tb, o, c, bi):
            v0s = jnp.broadcast_to(
                jnp.minimum(c * VC, VLAST), (L,)).astype(jnp.int32)
            lo16 = bscal(c) // L
            hi16 = (bscal(NCH + c) + L - 1) // L

            def vec_body(vec, _):
                s = pl.ds(vec * L, L)
                vb = sv_v[s]
                off = vb - v0s
                inb = (off >= 0) & (off < VC)
                offc = jnp.clip(off, 0, VC - 1)
                m16 = m_v[tb, s]
                a16 = a_v[tb, s]
                rowb = jnp.broadcast_to(bi * KG, (L,)).astype(jnp.int32)
                kkb = jnp.broadcast_to(k0 + o * KG, (L,)).astype(jnp.int32)
                for i in range(KG):
                    row = rowb + jnp.full((L,), i, jnp.int32)
                    x = plsc.load_gather(blk_v, [row, offc])
                    upd = (x > m16) & inb
                    m16 = jnp.where(upd, x, m16)
                    a16 = jnp.where(upd, kkb + jnp.full((L,), i, jnp.int32),
                                    a16)
                m_v[tb, s] = m16
                a_v[tb, s] = a16
                return 0

            lax.fori_loop(lo16, hi16, vec_body, 0)

        def scan_tail(tb, o):
            v0s = jnp.full((L,), VMAIN, jnp.int32)
            lo16 = bscal(2 * NCH) // L
            hi16 = (bscal(2 * NCH + 1) + L - 1) // L

            def vec_body(vec, _):
                s = pl.ds(vec * L, L)
                vb = sv_v[s]
                off = vb - v0s
                inb = off >= 0
                offc = jnp.clip(off, 0, TAIL - 1)
                m16 = m_v[tb, s]
                a16 = a_v[tb, s]
                kkb = jnp.broadcast_to(k0 + o * KG, (L,)).astype(jnp.int32)
                for i in range(KG):
                    krow = kkb + jnp.full((L,), i, jnp.int32)
                    x = plsc.load_gather(tail_v, [krow, offc])
                    upd = (x > m16) & inb
                    m16 = jnp.where(upd, x, m16)
                    a16 = jnp.where(upd, krow, a16)
                m_v[tb, s] = m16
                a_v[tb, s] = a16
                return 0

            lax.fori_loop(lo16, hi16, vec_body, 0)

        for tb, tab_hbm in ((0, c_hbm), (1, h_hbm)):
            pltpu.sync_copy(tails_hbm.at[tb, d], tail_v)
            for o in range(NKG):
                issue(tab_hbm, o, jnp.int32(0), jnp.int32(0))

                def chunk_body(c, _, tb=tb, o=o, tab_hbm=tab_hbm):
                    bi = c % 2

                    @pl.when(c + 1 < NCH)
                    def _():
                        issue(tab_hbm, o, c + 1, (c + 1) % 2)

                    drain(bi)
                    scan_chunk(tb, o, c, bi)
                    return 0

                lax.fori_loop(0, NCH, chunk_body, 0)
                scan_tail(tb, o)

        pltpu.sync_copy(m_v, shf_v.at[sid])
        pltpu.sync_copy(a_v, shi_v.at[sid])
        plsc.subcore_barrier()

        @pl.when(kh == 0)
        def _():
            for tb in range(2):
                pltpu.sync_copy(shf_v.at[sid + 1, tb], pm_v)
                pltpu.sync_copy(shi_v.at[sid + 1, tb], pa_v)

                def merge_body(i, _, tb=tb):
                    s = pl.ds(i * L, L)
                    upd = pm_v[s] > m_v[tb, s]
                    a_v[tb, s] = jnp.where(upd, pa_v[s], a_v[tb, s])
                    return 0

                lax.fori_loop(0, B // L, merge_body, 0)

            def out_body(i, _):
                s = pl.ds(i * L, L)
                plsc.store_scatter(row_v, [ord_v[s]], a_v[0, s] + a_v[1, s])
                return 0

            lax.fori_loop(0, B // L, out_body, 0)
            pltpu.sync_copy(row_v, out_hbm.at[d])

    out_t = sc_kernel(sv, order, bounds, vt_c, vt_h, tails)
    return out_t.T


# speedup vs baseline: 5.5576x; 1.0200x over previous
import functools

import jax
import jax.numpy as jnp
from jax import lax
from jax.experimental import pallas as pl
from jax.experimental.pallas import tpu as pltpu
from jax.experimental.pallas import tpu_sc as plsc

D = 16
K = 32
NC, NS, L = 2, 16, 16
NBUF = 2
KG = 8
KH = K // 2
NKG = KH // KG
VC = 4096
VMAIN = 99968
NCH = 25
VLAST = VMAIN - VC


def kernel(voc_idxs, pai_concept, pai_character):
    B = voc_idxs.shape[0]
    V = pai_concept.shape[0]
    TAIL = V - VMAIN

    vt_c = pai_concept.transpose(1, 2, 0)
    vt_h = pai_character.transpose(1, 2, 0)
    tails = jnp.stack([vt_c[:, :, VMAIN:], vt_h[:, :, VMAIN:]])

    order = jnp.argsort(voc_idxs).astype(jnp.int32)
    sv = voc_idxs[order]
    grid = jnp.minimum(jnp.arange(NCH, dtype=jnp.int32) * VC, VLAST)
    cuts = jnp.concatenate([grid, grid + VC, jnp.array([VMAIN], jnp.int32),
                            jnp.full((64 - 2 * NCH - 1,), V, jnp.int32)])
    bounds = jnp.sum(sv[None, :] < cuts[:, None], axis=1,
                     dtype=jnp.int32)

    mesh = plsc.VectorSubcoreMesh(core_axis_name="c", subcore_axis_name="s")

    @functools.partial(
        pl.kernel,
        out_type=jax.ShapeDtypeStruct((D, B), jnp.int32),
        mesh=mesh,
        scratch_types=[
            pltpu.VMEM((64,), jnp.int32),
            pltpu.VMEM((B,), jnp.int32),
            pltpu.VMEM((B,), jnp.int32),
            pltpu.VMEM((NBUF * KG, VC), jnp.float32),
            pltpu.VMEM((K, TAIL), jnp.float32),
            pltpu.VMEM((2, B), jnp.float32),
            pltpu.VMEM((2, B), jnp.int32),
            pltpu.VMEM((B,), jnp.float32),
            pltpu.VMEM((B,), jnp.int32),
            pltpu.VMEM((B,), jnp.int32),
            pltpu.VMEM_SHARED((NS, 2, B), jnp.float32),
            pltpu.VMEM_SHARED((NS, 2, B), jnp.int32),
            pltpu.SemaphoreType.DMA((4, NBUF)),
        ],
        compiler_params=pltpu.CompilerParams(needs_layout_passes=False),
    )
    def sc_kernel(sv_hbm, order_hbm, bounds_hbm, c_hbm, h_hbm, tails_hbm,
                  out_hbm, bnd_v, sv_v, ord_v, blk_v, tail_v,
                  m_v, a_v, pm_v, pa_v, row_v, shf_v, shi_v, sem):
        cid = lax.axis_index("c")
        sid = lax.axis_index("s")
        d = (sid // 2) * NC + cid
        kh = sid % 2
        k0 = kh * KH

        pltpu.sync_copy(bounds_hbm, bnd_v)
        pltpu.sync_copy(sv_hbm, sv_v)
        pltpu.sync_copy(order_hbm, ord_v)

        neg_inf = jnp.full((L,), -jnp.inf, jnp.float32)
        zeros = jnp.zeros((L,), jnp.int32)

        def init_body(i, _):
            s = pl.ds(i * L, L)
            for tb in range(2):
                m_v[tb, s] = neg_inf
                a_v[tb, s] = zeros
            return 0

        lax.fori_loop(0, B // L, init_body, 0)

        def bscal(i):
            return plsc.load_gather(bnd_v, [jnp.full((L,), i, jnp.int32)])[0]

        HV = VC // 4

        def issue(tab_hbm, o, c, bi):
            v0 = pl.multiple_of(jnp.minimum(c * VC, VLAST), 128)
            row0 = pl.multiple_of(bi * KG, KG)
            krow = pl.multiple_of(k0 + o * KG, KG)
            for q in range(4):
                vq = pl.multiple_of(v0 + q * HV, 128)
                pltpu.async_copy(
                    tab_hbm.at[d, pl.ds(krow, KG), pl.ds(vq, HV)],
                    blk_v.at[pl.ds(row0, KG), pl.ds(q * HV, HV)],
                    sem.at[q, bi])

        def drain(bi):
            dummy = c_hbm.at[0, pl.ds(0, KG), pl.ds(0, HV)]
            for q in range(4):
                pltpu.make_async_copy(
                    dummy, blk_v.at[pl.ds(0, KG), pl.ds(0, HV)],
                    sem.at[q, bi]).wait()

        def scan_chunk(tb, o, c, bi):
            v0s = jnp.broadcast_to(
                jnp.minimum(c * VC, VLAST), (L,)).astype(jnp.int32)
            lo16 = bscal(c) // L
            hi16 = (bscal(NCH + c) + L - 1) // L

            def vec_body(vec, _):
                s = pl.ds(vec * L, L)
                vb = sv_v[s]
                off = vb - v0s
                inb = (off >= 0) & (off < VC)
                offc = jnp.clip(off, 0, VC - 1)
                m16 = m_v[tb, s]
                a16 = a_v[tb, s]
                rowb = jnp.broadcast_to(bi * KG, (L,)).astype(jnp.int32)
                kkb = jnp.broadcast_to(k0 + o * KG, (L,)).astype(jnp.int32)
                for i in range(KG):
                    row = rowb + jnp.full((L,), i, jnp.int32)
                    x = plsc.load_gather(blk_v, [row, offc])
                    upd = (x > m16) & inb
                    m16 = jnp.where(upd, x, m16)
                    a16 = jnp.where(upd, kkb + jnp.full((L,), i, jnp.int32),
                                    a16)
                m_v[tb, s] = m16
                a_v[tb, s] = a16
                return 0

            lax.fori_loop(lo16, hi16, vec_body, 0)

        def scan_tail(tb, o):
            v0s = jnp.full((L,), VMAIN, jnp.int32)
            lo16 = bscal(2 * NCH) // L
            hi16 = (bscal(2 * NCH + 1) + L - 1) // L

            def vec_body(vec, _):
                s = pl.ds(vec * L, L)
                vb = sv_v[s]
                off = vb - v0s
                inb = off >= 0
                offc = jnp.clip(off, 0, TAIL - 1)
                m16 = m_v[tb, s]
                a16 = a_v[tb, s]
                kkb = jnp.broadcast_to(k0 + o * KG, (L,)).astype(jnp.int32)
                for i in range(KG):
                    krow = kkb + jnp.full((L,), i, jnp.int32)
                    x = plsc.load_gather(tail_v, [krow, offc])
                    upd = (x > m16) & inb
                    m16 = jnp.where(upd, x, m16)
                    a16 = jnp.where(upd, krow, a16)
                m_v[tb, s] = m16
                a_v[tb, s] = a16
                return 0

            lax.fori_loop(lo16, hi16, vec_body, 0)

        for tb, tab_hbm in ((0, c_hbm), (1, h_hbm)):
            pltpu.sync_copy(tails_hbm.at[tb, d], tail_v)
            for o in range(NKG):
                issue(tab_hbm, o, jnp.int32(0), jnp.int32(0))

                def chunk_body(c, _, tb=tb, o=o, tab_hbm=tab_hbm):
                    bi = c % 2

                    @pl.when(c + 1 < NCH)
                    def _():
                        issue(tab_hbm, o, c + 1, (c + 1) % 2)

                    drain(bi)
                    scan_chunk(tb, o, c, bi)
                    return 0

                lax.fori_loop(0, NCH, chunk_body, 0)
                scan_tail(tb, o)

        pltpu.sync_copy(m_v, shf_v.at[sid])
        pltpu.sync_copy(a_v, shi_v.at[sid])
        plsc.subcore_barrier()

        @pl.when(kh == 0)
        def _():
            for tb in range(2):
                pltpu.sync_copy(shf_v.at[sid + 1, tb], pm_v)
                pltpu.sync_copy(shi_v.at[sid + 1, tb], pa_v)

                def merge_body(i, _, tb=tb):
                    s = pl.ds(i * L, L)
                    upd = pm_v[s] > m_v[tb, s]
                    a_v[tb, s] = jnp.where(upd, pa_v[s], a_v[tb, s])
                    return 0

                lax.fori_loop(0, B // L, merge_body, 0)

            def out_body(i, _):
                s = pl.ds(i * L, L)
                plsc.store_scatter(row_v, [ord_v[s]], a_v[0, s] + a_v[1, s])
                return 0

            lax.fori_loop(0, B // L, out_body, 0)
            pltpu.sync_copy(row_v, out_hbm.at[d])

    out_t = sc_kernel(sv, order, bounds, vt_c, vt_h, tails)
    return out_t.T


# speedup vs baseline: 5.7379x; 1.0324x over previous
import functools

import jax
import jax.numpy as jnp
from jax import lax
from jax.experimental import pallas as pl
from jax.experimental.pallas import tpu as pltpu
from jax.experimental.pallas import tpu_sc as plsc

D = 16
K = 32
NC, NS, L = 2, 16, 16
NBUF = 2
KG = 8
KH = K // 2
NKG = KH // KG
VC = 4096
VMAIN = 99968
NCH = 25
VLAST = VMAIN - VC


def kernel(voc_idxs, pai_concept, pai_character):
    B = voc_idxs.shape[0]
    V = pai_concept.shape[0]
    TAIL = V - VMAIN

    vt_c = pai_concept.transpose(1, 2, 0)
    vt_h = pai_character.transpose(1, 2, 0)
    tails = jnp.stack([vt_c[:, :, VMAIN:], vt_h[:, :, VMAIN:]])

    sv, order = lax.sort(
        (voc_idxs, lax.iota(jnp.int32, B)), num_keys=1)
    grid = jnp.minimum(jnp.arange(NCH, dtype=jnp.int32) * VC, VLAST)
    cuts = jnp.concatenate([grid, grid + VC, jnp.array([VMAIN], jnp.int32),
                            jnp.full((64 - 2 * NCH - 1,), V, jnp.int32)])
    bounds = jnp.sum(sv[None, :] < cuts[:, None], axis=1,
                     dtype=jnp.int32)

    mesh = plsc.VectorSubcoreMesh(core_axis_name="c", subcore_axis_name="s")

    @functools.partial(
        pl.kernel,
        out_type=jax.ShapeDtypeStruct((D, B), jnp.int32),
        mesh=mesh,
        scratch_types=[
            pltpu.VMEM((64,), jnp.int32),
            pltpu.VMEM((B,), jnp.int32),
            pltpu.VMEM((B,), jnp.int32),
            pltpu.VMEM((NBUF * KG, VC), jnp.float32),
            pltpu.VMEM((K, TAIL), jnp.float32),
            pltpu.VMEM((2, B), jnp.float32),
            pltpu.VMEM((2, B), jnp.int32),
            pltpu.VMEM((B,), jnp.float32),
            pltpu.VMEM((B,), jnp.int32),
            pltpu.VMEM((B,), jnp.int32),
            pltpu.VMEM_SHARED((NS, 2, B), jnp.float32),
            pltpu.VMEM_SHARED((NS, 2, B), jnp.int32),
            pltpu.SemaphoreType.DMA((4, NBUF)),
        ],
        compiler_params=pltpu.CompilerParams(needs_layout_passes=False),
    )
    def sc_kernel(sv_hbm, order_hbm, bounds_hbm, c_hbm, h_hbm, tails_hbm,
                  out_hbm, bnd_v, sv_v, ord_v, blk_v, tail_v,
                  m_v, a_v, pm_v, pa_v, row_v, shf_v, shi_v, sem):
        cid = lax.axis_index("c")
        sid = lax.axis_index("s")
        d = (sid // 2) * NC + cid
        kh = sid % 2
        k0 = kh * KH

        pltpu.sync_copy(bounds_hbm, bnd_v)
        pltpu.sync_copy(sv_hbm, sv_v)
        pltpu.sync_copy(order_hbm, ord_v)

        neg_inf = jnp.full((L,), -jnp.inf, jnp.float32)
        zeros = jnp.zeros((L,), jnp.int32)

        def init_body(i, _):
            s = pl.ds(i * L, L)
            for tb in range(2):
                m_v[tb, s] = neg_inf
                a_v[tb, s] = zeros
            return 0

        lax.fori_loop(0, B // L, init_body, 0)

        def bscal(i):
            return plsc.load_gather(bnd_v, [jnp.full((L,), i, jnp.int32)])[0]

        HV = VC // 4

        def issue(tab_hbm, o, c, bi):
            v0 = pl.multiple_of(jnp.minimum(c * VC, VLAST), 128)
            row0 = pl.multiple_of(bi * KG, KG)
            krow = pl.multiple_of(k0 + o * KG, KG)
            for q in range(4):
                vq = pl.multiple_of(v0 + q * HV, 128)
                pltpu.async_copy(
                    tab_hbm.at[d, pl.ds(krow, KG), pl.ds(vq, HV)],
                    blk_v.at[pl.ds(row0, KG), pl.ds(q * HV, HV)],
                    sem.at[q, bi])

        def drain(bi):
            dummy = c_hbm.at[0, pl.ds(0, KG), pl.ds(0, HV)]
            for q in range(4):
                pltpu.make_async_copy(
                    dummy, blk_v.at[pl.ds(0, KG), pl.ds(0, HV)],
                    sem.at[q, bi]).wait()

        def scan_chunk(tb, o, c, bi):
            v0s = jnp.broadcast_to(
                jnp.minimum(c * VC, VLAST), (L,)).astype(jnp.int32)
            lo16 = bscal(c) // L
            hi16 = (bscal(NCH + c) + L - 1) // L

            def vec_body(vec, _):
                s = pl.ds(vec * L, L)
                vb = sv_v[s]
                off = vb - v0s
                inb = (off >= 0) & (off < VC)
                offc = jnp.clip(off, 0, VC - 1)
                m16 = m_v[tb, s]
                a16 = a_v[tb, s]
                rowb = jnp.broadcast_to(bi * KG, (L,)).astype(jnp.int32)
                kkb = jnp.broadcast_to(k0 + o * KG, (L,)).astype(jnp.int32)
                for i in range(KG):
                    row = rowb + jnp.full((L,), i, jnp.int32)
                    x = plsc.load_gather(blk_v, [row, offc])
                    upd = (x > m16) & inb
                    m16 = jnp.where(upd, x, m16)
                    a16 = jnp.where(upd, kkb + jnp.full((L,), i, jnp.int32),
                                    a16)
                m_v[tb, s] = m16
                a_v[tb, s] = a16
                return 0

            lax.fori_loop(lo16, hi16, vec_body, 0)

        def scan_tail(tb, o):
            v0s = jnp.full((L,), VMAIN, jnp.int32)
            lo16 = bscal(2 * NCH) // L
            hi16 = (bscal(2 * NCH + 1) + L - 1) // L

            def vec_body(vec, _):
                s = pl.ds(vec * L, L)
                vb = sv_v[s]
                off = vb - v0s
                inb = off >= 0
                offc = jnp.clip(off, 0, TAIL - 1)
                m16 = m_v[tb, s]
                a16 = a_v[tb, s]
                kkb = jnp.broadcast_to(k0 + o * KG, (L,)).astype(jnp.int32)
                for i in range(KG):
                    krow = kkb + jnp.full((L,), i, jnp.int32)
                    x = plsc.load_gather(tail_v, [krow, offc])
                    upd = (x > m16) & inb
                    m16 = jnp.where(upd, x, m16)
                    a16 = jnp.where(upd, krow, a16)
                m_v[tb, s] = m16
                a_v[tb, s] = a16
                return 0

            lax.fori_loop(lo16, hi16, vec_body, 0)

        for tb, tab_hbm in ((0, c_hbm), (1, h_hbm)):
            pltpu.sync_copy(tails_hbm.at[tb, d], tail_v)
            for o in range(NKG):
                issue(tab_hbm, o, jnp.int32(0), jnp.int32(0))

                def chunk_body(c, _, tb=tb, o=o, tab_hbm=tab_hbm):
                    bi = c % 2

                    @pl.when(c + 1 < NCH)
                    def _():
                        issue(tab_hbm, o, c + 1, (c + 1) % 2)

                    drain(bi)
                    scan_chunk(tb, o, c, bi)
                    return 0

                lax.fori_loop(0, NCH, chunk_body, 0)
                scan_tail(tb, o)

        pltpu.sync_copy(m_v, shf_v.at[sid])
        pltpu.sync_copy(a_v, shi_v.at[sid])
        plsc.subcore_barrier()

        @pl.when(kh == 0)
        def _():
            for tb in range(2):
                pltpu.sync_copy(shf_v.at[sid + 1, tb], pm_v)
                pltpu.sync_copy(shi_v.at[sid + 1, tb], pa_v)

                def merge_body(i, _, tb=tb):
                    s = pl.ds(i * L, L)
                    upd = pm_v[s] > m_v[tb, s]
                    a_v[tb, s] = jnp.where(upd, pa_v[s], a_v[tb, s])
                    return 0

                lax.fori_loop(0, B // L, merge_body, 0)

            def out_body(i, _):
                s = pl.ds(i * L, L)
                plsc.store_scatter(row_v, [ord_v[s]], a_v[0, s] + a_v[1, s])
                return 0

            lax.fori_loop(0, B // L, out_body, 0)
            pltpu.sync_copy(row_v, out_hbm.at[d])

    out_t = sc_kernel(sv, order, bounds, vt_c, vt_h, tails)
    return out_t.T


# speedup vs baseline: 5.8234x; 1.0149x over previous
import functools

import jax
import jax.numpy as jnp
from jax import lax
from jax.experimental import pallas as pl
from jax.experimental.pallas import tpu as pltpu
from jax.experimental.pallas import tpu_sc as plsc

D = 16
K = 32
NC, NS, L = 2, 16, 16
NBUF = 2
KG = 8
KH = K // 2
NKG = KH // KG
VC = 4096
VMAIN = 99968
NCH = 25
VLAST = VMAIN - VC


def kernel(voc_idxs, pai_concept, pai_character):
    B = voc_idxs.shape[0]
    V = pai_concept.shape[0]
    TAIL = V - VMAIN

    vt_c = pai_concept.transpose(1, 2, 0)
    vt_h = pai_character.transpose(1, 2, 0)
    tails = jnp.stack([vt_c[:, :, VMAIN:], vt_h[:, :, VMAIN:]])

    sv, order = lax.sort(
        (voc_idxs, lax.iota(jnp.int32, B)), num_keys=1)
    grid = jnp.minimum(jnp.arange(NCH, dtype=jnp.int32) * VC, VLAST)
    cuts = jnp.concatenate([grid, grid + VC, jnp.array([VMAIN], jnp.int32),
                            jnp.full((64 - 2 * NCH - 1,), V, jnp.int32)])
    bounds = jnp.sum(sv[None, :] < cuts[:, None], axis=1,
                     dtype=jnp.int32)

    mesh = plsc.VectorSubcoreMesh(core_axis_name="c", subcore_axis_name="s")

    @functools.partial(
        pl.kernel,
        out_type=jax.ShapeDtypeStruct((D, B), jnp.int32),
        mesh=mesh,
        scratch_types=[
            pltpu.VMEM((64,), jnp.int32),
            pltpu.VMEM((B,), jnp.int32),
            pltpu.VMEM((B,), jnp.int32),
            pltpu.VMEM((NBUF * KG, VC), jnp.float32),
            pltpu.VMEM((K, TAIL), jnp.float32),
            pltpu.VMEM((2, B), jnp.float32),
            pltpu.VMEM((2, B), jnp.int32),
            pltpu.VMEM((B,), jnp.float32),
            pltpu.VMEM((B,), jnp.int32),
            pltpu.VMEM((B,), jnp.int32),
            pltpu.VMEM_SHARED((NS, 2, B), jnp.float32),
            pltpu.VMEM_SHARED((NS, 2, B), jnp.int32),
            pltpu.SemaphoreType.DMA((8, NBUF)),
        ],
        compiler_params=pltpu.CompilerParams(needs_layout_passes=False),
    )
    def sc_kernel(sv_hbm, order_hbm, bounds_hbm, c_hbm, h_hbm, tails_hbm,
                  out_hbm, bnd_v, sv_v, ord_v, blk_v, tail_v,
                  m_v, a_v, pm_v, pa_v, row_v, shf_v, shi_v, sem):
        cid = lax.axis_index("c")
        sid = lax.axis_index("s")
        d = (sid // 2) * NC + cid
        kh = sid % 2
        k0 = kh * KH

        pltpu.sync_copy(bounds_hbm, bnd_v)
        pltpu.sync_copy(sv_hbm, sv_v)
        pltpu.sync_copy(order_hbm, ord_v)

        neg_inf = jnp.full((L,), -jnp.inf, jnp.float32)
        zeros = jnp.zeros((L,), jnp.int32)

        def init_body(i, _):
            s = pl.ds(i * L, L)
            for tb in range(2):
                m_v[tb, s] = neg_inf
                a_v[tb, s] = zeros
            return 0

        lax.fori_loop(0, B // L, init_body, 0)

        def bscal(i):
            return plsc.load_gather(bnd_v, [jnp.full((L,), i, jnp.int32)])[0]

        HV = VC // 8

        def issue(tab_hbm, o, c, bi):
            v0 = pl.multiple_of(jnp.minimum(c * VC, VLAST), 128)
            row0 = pl.multiple_of(bi * KG, KG)
            krow = pl.multiple_of(k0 + o * KG, KG)
            for q in range(8):
                vq = pl.multiple_of(v0 + q * HV, 128)
                pltpu.async_copy(
                    tab_hbm.at[d, pl.ds(krow, KG), pl.ds(vq, HV)],
                    blk_v.at[pl.ds(row0, KG), pl.ds(q * HV, HV)],
                    sem.at[q, bi])

        def drain(bi):
            dummy = c_hbm.at[0, pl.ds(0, KG), pl.ds(0, HV)]
            for q in range(8):
                pltpu.make_async_copy(
                    dummy, blk_v.at[pl.ds(0, KG), pl.ds(0, HV)],
                    sem.at[q, bi]).wait()

        def scan_chunk(tb, o, c, bi):
            v0s = jnp.broadcast_to(
                jnp.minimum(c * VC, VLAST), (L,)).astype(jnp.int32)
            lo16 = bscal(c) // L
            hi16 = (bscal(NCH + c) + L - 1) // L

            def vec_body(vec, _):
                s = pl.ds(vec * L, L)
                vb = sv_v[s]
                off = vb - v0s
                inb = (off >= 0) & (off < VC)
                offc = jnp.clip(off, 0, VC - 1)
                m16 = m_v[tb, s]
                a16 = a_v[tb, s]
                rowb = jnp.broadcast_to(bi * KG, (L,)).astype(jnp.int32)
                kkb = jnp.broadcast_to(k0 + o * KG, (L,)).astype(jnp.int32)
                for i in range(KG):
                    row = rowb + jnp.full((L,), i, jnp.int32)
                    x = plsc.load_gather(blk_v, [row, offc])
                    upd = (x > m16) & inb
                    m16 = jnp.where(upd, x, m16)
                    a16 = jnp.where(upd, kkb + jnp.full((L,), i, jnp.int32),
                                    a16)
                m_v[tb, s] = m16
                a_v[tb, s] = a16
                return 0

            lax.fori_loop(lo16, hi16, vec_body, 0)

        def scan_tail(tb, o):
            v0s = jnp.full((L,), VMAIN, jnp.int32)
            lo16 = bscal(2 * NCH) // L
            hi16 = (bscal(2 * NCH + 1) + L - 1) // L

            def vec_body(vec, _):
                s = pl.ds(vec * L, L)
                vb = sv_v[s]
                off = vb - v0s
                inb = off >= 0
                offc = jnp.clip(off, 0, TAIL - 1)
                m16 = m_v[tb, s]
                a16 = a_v[tb, s]
                kkb = jnp.broadcast_to(k0 + o * KG, (L,)).astype(jnp.int32)
                for i in range(KG):
                    krow = kkb + jnp.full((L,), i, jnp.int32)
                    x = plsc.load_gather(tail_v, [krow, offc])
                    upd = (x > m16) & inb
                    m16 = jnp.where(upd, x, m16)
                    a16 = jnp.where(upd, krow, a16)
                m_v[tb, s] = m16
                a_v[tb, s] = a16
                return 0

            lax.fori_loop(lo16, hi16, vec_body, 0)

        for tb, tab_hbm in ((0, c_hbm), (1, h_hbm)):
            pltpu.sync_copy(tails_hbm.at[tb, d], tail_v)
            for o in range(NKG):
                issue(tab_hbm, o, jnp.int32(0), jnp.int32(0))

                def chunk_body(c, _, tb=tb, o=o, tab_hbm=tab_hbm):
                    bi = c % 2

                    @pl.when(c + 1 < NCH)
                    def _():
                        issue(tab_hbm, o, c + 1, (c + 1) % 2)

                    drain(bi)
                    scan_chunk(tb, o, c, bi)
                    return 0

                lax.fori_loop(0, NCH, chunk_body, 0)
                scan_tail(tb, o)

        pltpu.sync_copy(m_v, shf_v.at[sid])
        pltpu.sync_copy(a_v, shi_v.at[sid])
        plsc.subcore_barrier()

        @pl.when(kh == 0)
        def _():
            for tb in range(2):
                pltpu.sync_copy(shf_v.at[sid + 1, tb], pm_v)
                pltpu.sync_copy(shi_v.at[sid + 1, tb], pa_v)

                def merge_body(i, _, tb=tb):
                    s = pl.ds(i * L, L)
                    upd = pm_v[s] > m_v[tb, s]
                    a_v[tb, s] = jnp.where(upd, pa_v[s], a_v[tb, s])
                    return 0

                lax.fori_loop(0, B // L, merge_body, 0)

            def out_body(i, _):
                s = pl.ds(i * L, L)
                plsc.store_scatter(row_v, [ord_v[s]], a_v[0, s] + a_v[1, s])
                return 0

            lax.fori_loop(0, B // L, out_body, 0)
            pltpu.sync_copy(row_v, out_hbm.at[d])

    out_t = sc_kernel(sv, order, bounds, vt_c, vt_h, tails)
    return out_t.T
